# Initial kernel scaffold; baseline (speedup 1.0000x reference)
import jax
import jax.numpy as jnp
from jax.experimental import pallas as pl


def kernel(pairs, adj_entity_np, adj_relation_np, entity_emb, relation_emb, user_emb, W_w, W_b):
    raise NotImplementedError("write your pallas kernel here")


# speedup vs baseline: 2.7060x; 2.7060x over previous
import jax
import jax.numpy as jnp
from jax import lax
from jax.experimental import pallas as pl
from jax.experimental.pallas import tpu as pltpu
from jax.experimental.pallas import tpu_sc as plsc

_D = 32
_NN = 16
_B = 4096
_NR = 32
_NC = 2
_NS = 16
_NW = _NC * _NS
_PPW = _B // _NW
_SUB = 16
_H1S = (_PPW // _SUB) * _NN
_H2S = _H1S * _NN
_H1W = _PPW * _NN


def _sc_body(users_h, items_h, adj_e_h, adj_r_h, ent_h, usr_h,
             urows_o, e0_o, adjr0_o, e1_o, adjr1_o, e2_o,
             users_v, items_v, urow_v, e0_v, adje0_v, adjr0_v,
             idx1_v, e1_v, adje1_v, adjr1_v, idx2_v, e2_v, sem):
    wid = lax.axis_index("s") * _NC + lax.axis_index("c")
    pbase = wid * _PPW

    pltpu.sync_copy(users_h.at[pl.ds(pbase, _PPW)], users_v)
    pltpu.sync_copy(items_h.at[pl.ds(pbase, _PPW)], items_v)
    cu = pltpu.async_copy(usr_h.at[users_v], urow_v, sem)
    ce = pltpu.async_copy(ent_h.at[items_v], e0_v, sem)
    ca = pltpu.async_copy(adj_e_h.at[items_v], adje0_v, sem)
    cr = pltpu.async_copy(adj_r_h.at[items_v], adjr0_v, sem)
    cu.wait()
    ce.wait()
    ca.wait()
    cr.wait()
    pltpu.sync_copy(urow_v, urows_o.at[pl.ds(pbase, _PPW)])
    pltpu.sync_copy(e0_v, e0_o.at[pl.ds(pbase, _PPW)])
    pltpu.sync_copy(adjr0_v, adjr0_o.at[pl.ds(pbase, _PPW)])

    def _flat1(t, c):
        idx1_v[pl.ds(t * _NN, _NN)] = adje0_v[t, :]
        return c
    lax.fori_loop(0, _PPW, _flat1, 0)

    def _sub(s, c):
        h1base = pbase * _NN + s * _H1S
        idx1s = idx1_v.at[pl.ds(s * _H1S, _H1S)]
        c1 = pltpu.async_copy(ent_h.at[idx1s], e1_v, sem)
        c2 = pltpu.async_copy(adj_e_h.at[idx1s], adje1_v, sem)
        c3 = pltpu.async_copy(adj_r_h.at[idx1s], adjr1_v, sem)
        c1.wait()
        c2.wait()
        c3.wait()

        def _flat2(t, cc):
            idx2_v[pl.ds(t * _NN, _NN)] = adje1_v[t, :]
            return cc
        lax.fori_loop(0, _H1S, _flat2, 0)

        copies = []
        for ch in range(_H2S // 128):
            copies.append(pltpu.async_copy(
                ent_h.at[idx2_v.at[pl.ds(ch * 128, 128)]],
                e2_v.at[pl.ds(ch * 128, 128)], sem))
        pltpu.sync_copy(e1_v, e1_o.at[pl.ds(h1base, _H1S)])
        pltpu.sync_copy(adjr1_v, adjr1_o.at[pl.ds(h1base, _H1S)])
        for cp in copies:
            cp.wait()
        pltpu.sync_copy(e2_v, e2_o.at[pl.ds(h1base * _NN, _H2S)])
        return c
    lax.fori_loop(0, _SUB, _sub, 0)


_sc_gather = pl.kernel(
    _sc_body,
    out_type=[
        jax.ShapeDtypeStruct((_B, _D), jnp.float32),
        jax.ShapeDtypeStruct((_B, _D), jnp.float32),
        jax.ShapeDtypeStruct((_B, _NN), jnp.int32),
        jax.ShapeDtypeStruct((_B * _NN, _D), jnp.float32),
        jax.ShapeDtypeStruct((_B * _NN, _NN), jnp.int32),
        jax.ShapeDtypeStruct((_B * _NN * _NN, _D), jnp.float32),
    ],
    mesh=plsc.VectorSubcoreMesh(core_axis_name="c", subcore_axis_name="s"),
    compiler_params=pltpu.CompilerParams(use_tc_tiling_on_sc=False),
    scratch_types=[
        pltpu.VMEM((_PPW,), jnp.int32),
        pltpu.VMEM((_PPW,), jnp.int32),
        pltpu.VMEM((_PPW, _D), jnp.float32),
        pltpu.VMEM((_PPW, _D), jnp.float32),
        pltpu.VMEM((_PPW, _NN), jnp.int32),
        pltpu.VMEM((_PPW, _NN), jnp.int32),
        pltpu.VMEM((_H1W,), jnp.int32),
        pltpu.VMEM((_H1S, _D), jnp.float32),
        pltpu.VMEM((_H1S, _NN), jnp.int32),
        pltpu.VMEM((_H1S, _NN), jnp.int32),
        pltpu.VMEM((_H2S,), jnp.int32),
        pltpu.VMEM((_H2S, _D), jnp.float32),
        pltpu.SemaphoreType.DMA,
    ],
)

_P = 128
_G = _B // _P


def _tc_body(u_ref, e0_ref, r0_ref, e1_ref, r1_ref, e2_ref,
             rel_ref, ww_ref, wb_ref, out_ref):
    u = u_ref[...]
    rel = rel_ref[...]
    ww = ww_ref[...]
    wb = wb_ref[...]
    ur = lax.dot_general(u, rel, (((1,), (1,)), ((), ())),
                         preferred_element_type=jnp.float32)

    r0 = r0_ref[...]
    r1 = r1_ref[...]
    ur16 = jnp.broadcast_to(ur[:, None, :], (_P, _NN, _D)).reshape(_P * _NN, _D)
    s0 = jnp.zeros((_P, _NN), jnp.float32)
    s1 = jnp.zeros((_P * _NN, _NN), jnp.float32)
    for r in range(_NR):
        s0 = s0 + jnp.where(r0 == r, ur[:, r:r + 1], 0.0)
        s1 = s1 + jnp.where(r1 == r, ur16[:, r:r + 1], 0.0)

    w0 = jnp.exp(s0 - jnp.max(s0, axis=-1, keepdims=True))
    w0 = w0 / jnp.sum(w0, axis=-1, keepdims=True)
    w1 = jnp.exp(s1 - jnp.max(s1, axis=-1, keepdims=True))
    w1 = w1 / jnp.sum(w1, axis=-1, keepdims=True)

    e2 = e2_ref[...].reshape(_P * _NN, _NN, _D)
    m1 = jnp.sum(w1[:, :, None] * e2, axis=1)
    e1 = e1_ref[...]
    h1 = jnp.maximum(
        lax.dot_general(e1 + m1, ww, (((1,), (1,)), ((), ())),
                        preferred_element_type=jnp.float32) + wb, 0.0)

    e1p = e1.reshape(_P, _NN, _D)
    m0 = jnp.sum(w0[:, :, None] * e1p, axis=1)
    e0 = e0_ref[...]
    h0 = jnp.maximum(
        lax.dot_general(e0 + m0, ww, (((1,), (1,)), ((), ())),
                        preferred_element_type=jnp.float32) + wb, 0.0)

    mo = jnp.sum(w0[:, :, None] * h1.reshape(_P, _NN, _D), axis=1)
    o = jnp.tanh(
        lax.dot_general(h0 + mo, ww, (((1,), (1,)), ((), ())),
                        preferred_element_type=jnp.float32) + wb)
    pred = jnp.sum(u * o, axis=1)
    out_ref[...] = jax.nn.sigmoid(pred).reshape(1, 1, _P)


def _tc_call(urows, e0, adjr0, e1, adjr1, e2, rel, ww, wb2):
    return pl.pallas_call(
        _tc_body,
        grid=(_G,),
        in_specs=[
            pl.BlockSpec((_P, _D), lambda i: (i, 0)),
            pl.BlockSpec((_P, _D), lambda i: (i, 0)),
            pl.BlockSpec((_P, _NN), lambda i: (i, 0)),
            pl.BlockSpec((_P * _NN, _D), lambda i: (i, 0)),
            pl.BlockSpec((_P * _NN, _NN), lambda i: (i, 0)),
            pl.BlockSpec((_P * _NN * _NN, _D), lambda i: (i, 0)),
            pl.BlockSpec((_NR, _D), lambda i: (0, 0)),
            pl.BlockSpec((_D, _D), lambda i: (0, 0)),
            pl.BlockSpec((1, _D), lambda i: (0, 0)),
        ],
        out_specs=pl.BlockSpec((1, 1, _P), lambda i: (i, 0, 0)),
        out_shape=jax.ShapeDtypeStruct((_G, 1, _P), jnp.float32),
    )(urows, e0, adjr0, e1, adjr1, e2, rel, ww, wb2)


def kernel(pairs, adj_entity_np, adj_relation_np, entity_emb, relation_emb,
           user_emb, W_w, W_b):
    users = pairs[:, 0].astype(jnp.int32)
    items = pairs[:, 1].astype(jnp.int32)
    urows, e0, adjr0, e1, adjr1, e2 = _sc_gather(
        users, items, adj_entity_np.astype(jnp.int32),
        adj_relation_np.astype(jnp.int32), entity_emb, user_emb)
    out = _tc_call(urows, e0, adjr0, e1, adjr1, e2,
                   relation_emb, W_w, W_b.reshape(1, _D))
    return out.reshape(_B)


# speedup vs baseline: 2.7181x; 1.0045x over previous
import jax
import jax.numpy as jnp
from jax import lax
from jax.experimental import pallas as pl
from jax.experimental.pallas import tpu as pltpu
from jax.experimental.pallas import tpu_sc as plsc

_D = 32
_NN = 16
_B = 4096
_NR = 32
_NC = 2
_NS = 16
_NW = _NC * _NS
_PPW = _B // _NW
_SUB = 16
_H1S = (_PPW // _SUB) * _NN
_H2S = _H1S * _NN
_H1W = _PPW * _NN


def _sc_body(users_h, items_h, adj_e_h, adj_r_h, ent_h, usr_h,
             urows_o, e0_o, adjr0_o, e1_o, adjr1_o, e2_o,
             users_v, items_v, urow_v, e0_v, adje0_v, adjr0_v,
             idx1_v, e1_v, adje1_v, adjr1_v, idx2_v, e2_v, sem):
    wid = lax.axis_index("s") * _NC + lax.axis_index("c")
    pbase = wid * _PPW

    pltpu.sync_copy(users_h.at[pl.ds(pbase, _PPW)], users_v)
    pltpu.sync_copy(items_h.at[pl.ds(pbase, _PPW)], items_v)
    cu = pltpu.async_copy(usr_h.at[users_v], urow_v, sem)
    ce = pltpu.async_copy(ent_h.at[items_v], e0_v, sem)
    ca = pltpu.async_copy(adj_e_h.at[items_v], adje0_v, sem)
    cr = pltpu.async_copy(adj_r_h.at[items_v], adjr0_v, sem)
    cu.wait()
    ce.wait()
    ca.wait()
    cr.wait()
    pltpu.sync_copy(urow_v, urows_o.at[pl.ds(pbase, _PPW)])
    pltpu.sync_copy(e0_v, e0_o.at[pl.ds(pbase, _PPW)])
    pltpu.sync_copy(adjr0_v, adjr0_o.at[pl.ds(pbase, _PPW)])

    def _flat1(t, c):
        idx1_v[pl.ds(t * _NN, _NN)] = adje0_v[t, :]
        return c
    lax.fori_loop(0, _PPW, _flat1, 0)

    _HH = _H1S // 2

    def _sub(s, c):
        h1base = pbase * _NN + s * _H1S
        idxa = idx1_v.at[pl.ds(s * _H1S, _HH)]
        idxb = idx1_v.at[pl.ds(s * _H1S + _HH, _HH)]
        a1 = pltpu.async_copy(ent_h.at[idxa], e1_v.at[pl.ds(0, _HH)], sem)
        a2 = pltpu.async_copy(adj_e_h.at[idxa], adje1_v.at[pl.ds(0, _HH)], sem)
        a3 = pltpu.async_copy(adj_r_h.at[idxa], adjr1_v.at[pl.ds(0, _HH)], sem)
        b1 = pltpu.async_copy(ent_h.at[idxb], e1_v.at[pl.ds(_HH, _HH)], sem)
        b2 = pltpu.async_copy(adj_e_h.at[idxb], adje1_v.at[pl.ds(_HH, _HH)], sem)
        b3 = pltpu.async_copy(adj_r_h.at[idxb], adjr1_v.at[pl.ds(_HH, _HH)], sem)
        a2.wait()

        def _flat2a(t, cc):
            idx2_v[pl.ds(t * _NN, _NN)] = adje1_v[t, :]
            return cc
        lax.fori_loop(0, _HH, _flat2a, 0)
        e2a = pltpu.async_copy(ent_h.at[idx2_v.at[pl.ds(0, _HH * _NN)]],
                               e2_v.at[pl.ds(0, _HH * _NN)], sem)
        b2.wait()

        def _flat2b(t, cc):
            idx2_v[pl.ds(t * _NN, _NN)] = adje1_v[t, :]
            return cc
        lax.fori_loop(_HH, _H1S, _flat2b, 0)
        e2b = pltpu.async_copy(ent_h.at[idx2_v.at[pl.ds(_HH * _NN, _HH * _NN)]],
                               e2_v.at[pl.ds(_HH * _NN, _HH * _NN)], sem)
        a1.wait()
        a3.wait()
        b1.wait()
        b3.wait()
        pltpu.sync_copy(e1_v, e1_o.at[pl.ds(h1base, _H1S)])
        pltpu.sync_copy(adjr1_v, adjr1_o.at[pl.ds(h1base, _H1S)])
        e2a.wait()
        e2b.wait()
        pltpu.sync_copy(e2_v, e2_o.at[pl.ds(h1base * _NN, _H2S)])
        return c
    lax.fori_loop(0, _SUB, _sub, 0)


_sc_gather = pl.kernel(
    _sc_body,
    out_type=[
        jax.ShapeDtypeStruct((_B, _D), jnp.float32),
        jax.ShapeDtypeStruct((_B, _D), jnp.float32),
        jax.ShapeDtypeStruct((_B, _NN), jnp.int32),
        jax.ShapeDtypeStruct((_B * _NN, _D), jnp.float32),
        jax.ShapeDtypeStruct((_B * _NN, _NN), jnp.int32),
        jax.ShapeDtypeStruct((_B * _NN * _NN, _D), jnp.float32),
    ],
    mesh=plsc.VectorSubcoreMesh(core_axis_name="c", subcore_axis_name="s"),
    compiler_params=pltpu.CompilerParams(use_tc_tiling_on_sc=False),
    scratch_types=[
        pltpu.VMEM((_PPW,), jnp.int32),
        pltpu.VMEM((_PPW,), jnp.int32),
        pltpu.VMEM((_PPW, _D), jnp.float32),
        pltpu.VMEM((_PPW, _D), jnp.float32),
        pltpu.VMEM((_PPW, _NN), jnp.int32),
        pltpu.VMEM((_PPW, _NN), jnp.int32),
        pltpu.VMEM((_H1W,), jnp.int32),
        pltpu.VMEM((_H1S, _D), jnp.float32),
        pltpu.VMEM((_H1S, _NN), jnp.int32),
        pltpu.VMEM((_H1S, _NN), jnp.int32),
        pltpu.VMEM((_H2S,), jnp.int32),
        pltpu.VMEM((_H2S, _D), jnp.float32),
        pltpu.SemaphoreType.DMA,
    ],
)

_P = 128
_G = _B // _P


def _tc_body(u_ref, e0_ref, r0_ref, e1_ref, r1_ref, e2_ref,
             rel_ref, ww_ref, wb_ref, out_ref):
    u = u_ref[...]
    rel = rel_ref[...]
    ww = ww_ref[...]
    wb = wb_ref[...]
    ur = lax.dot_general(u, rel, (((1,), (1,)), ((), ())),
                         preferred_element_type=jnp.float32)

    r0 = r0_ref[...]
    r1 = r1_ref[...]
    ur16 = jnp.broadcast_to(ur[:, None, :], (_P, _NN, _D)).reshape(_P * _NN, _D)
    s0 = jnp.zeros((_P, _NN), jnp.float32)
    s1 = jnp.zeros((_P * _NN, _NN), jnp.float32)
    for r in range(_NR):
        s0 = s0 + jnp.where(r0 == r, ur[:, r:r + 1], 0.0)
        s1 = s1 + jnp.where(r1 == r, ur16[:, r:r + 1], 0.0)

    w0 = jnp.exp(s0 - jnp.max(s0, axis=-1, keepdims=True))
    w0 = w0 / jnp.sum(w0, axis=-1, keepdims=True)
    w1 = jnp.exp(s1 - jnp.max(s1, axis=-1, keepdims=True))
    w1 = w1 / jnp.sum(w1, axis=-1, keepdims=True)

    e2 = e2_ref[...].reshape(_P * _NN, _NN, _D)
    m1 = jnp.sum(w1[:, :, None] * e2, axis=1)
    e1 = e1_ref[...]
    h1 = jnp.maximum(
        lax.dot_general(e1 + m1, ww, (((1,), (1,)), ((), ())),
                        preferred_element_type=jnp.float32) + wb, 0.0)

    e1p = e1.reshape(_P, _NN, _D)
    m0 = jnp.sum(w0[:, :, None] * e1p, axis=1)
    e0 = e0_ref[...]
    h0 = jnp.maximum(
        lax.dot_general(e0 + m0, ww, (((1,), (1,)), ((), ())),
                        preferred_element_type=jnp.float32) + wb, 0.0)

    mo = jnp.sum(w0[:, :, None] * h1.reshape(_P, _NN, _D), axis=1)
    o = jnp.tanh(
        lax.dot_general(h0 + mo, ww, (((1,), (1,)), ((), ())),
                        preferred_element_type=jnp.float32) + wb)
    pred = jnp.sum(u * o, axis=1)
    out_ref[...] = jax.nn.sigmoid(pred).reshape(1, 1, _P)


def _tc_call(urows, e0, adjr0, e1, adjr1, e2, rel, ww, wb2):
    return pl.pallas_call(
        _tc_body,
        grid=(_G,),
        in_specs=[
            pl.BlockSpec((_P, _D), lambda i: (i, 0)),
            pl.BlockSpec((_P, _D), lambda i: (i, 0)),
            pl.BlockSpec((_P, _NN), lambda i: (i, 0)),
            pl.BlockSpec((_P * _NN, _D), lambda i: (i, 0)),
            pl.BlockSpec((_P * _NN, _NN), lambda i: (i, 0)),
            pl.BlockSpec((_P * _NN * _NN, _D), lambda i: (i, 0)),
            pl.BlockSpec((_NR, _D), lambda i: (0, 0)),
            pl.BlockSpec((_D, _D), lambda i: (0, 0)),
            pl.BlockSpec((1, _D), lambda i: (0, 0)),
        ],
        out_specs=pl.BlockSpec((1, 1, _P), lambda i: (i, 0, 0)),
        out_shape=jax.ShapeDtypeStruct((_G, 1, _P), jnp.float32),
    )(urows, e0, adjr0, e1, adjr1, e2, rel, ww, wb2)


def kernel(pairs, adj_entity_np, adj_relation_np, entity_emb, relation_emb,
           user_emb, W_w, W_b):
    users = pairs[:, 0].astype(jnp.int32)
    items = pairs[:, 1].astype(jnp.int32)
    urows, e0, adjr0, e1, adjr1, e2 = _sc_gather(
        users, items, adj_entity_np.astype(jnp.int32),
        adj_relation_np.astype(jnp.int32), entity_emb, user_emb)
    out = _tc_call(urows, e0, adjr0, e1, adjr1, e2,
                   relation_emb, W_w, W_b.reshape(1, _D))
    return out.reshape(_B)


# speedup vs baseline: 3.8033x; 1.3993x over previous
import jax
import jax.numpy as jnp
from jax import lax
from jax.experimental import pallas as pl
from jax.experimental.pallas import tpu as pltpu
from jax.experimental.pallas import tpu_sc as plsc

_D = 32
_NN = 16
_B = 4096
_NR = 32
_NU = 100000
_NC = 2
_NS = 16
_NW = _NC * _NS
_PPW = _B // _NW
_SUB = 16
_PPS = _PPW // _SUB
_H1S = _PPS * _NN
_H2S = _H1S * _NN
_H1W = _PPW * _NN
_HH = _H1S // 2


def _sc_body(users_h, items_h, adj_e_h, adj_r_h, ent_h, usr_h, ur_h,
             urows_o, e0_o, adjr0_o, e1_o, m1_o,
             users_v, items_v, urow_v, e0_v, adje0_v, adjr0_v, ur_v,
             idx1_v, e1_v, adje1_v, adjr1_v, idx2_v, e2_v, m1_v, w_v, sem):
    wid = lax.axis_index("s") * _NC + lax.axis_index("c")
    pbase = wid * _PPW

    pltpu.sync_copy(users_h.at[pl.ds(pbase, _PPW)], users_v)
    pltpu.sync_copy(items_h.at[pl.ds(pbase, _PPW)], items_v)
    cu = pltpu.async_copy(usr_h.at[users_v], urow_v, sem)
    cg = pltpu.async_copy(ur_h.at[users_v], ur_v, sem)
    ce = pltpu.async_copy(ent_h.at[items_v], e0_v, sem)
    ca = pltpu.async_copy(adj_e_h.at[items_v], adje0_v, sem)
    cr = pltpu.async_copy(adj_r_h.at[items_v], adjr0_v, sem)
    cu.wait()
    cg.wait()
    ce.wait()
    ca.wait()
    cr.wait()
    pltpu.sync_copy(urow_v, urows_o.at[pl.ds(pbase, _PPW)])
    pltpu.sync_copy(e0_v, e0_o.at[pl.ds(pbase, _PPW)])
    pltpu.sync_copy(adjr0_v, adjr0_o.at[pl.ds(pbase, _PPW)])

    def _flat1(t, c):
        idx1_v[pl.ds(t * _NN, _NN)] = adje0_v[t, :]
        return c
    lax.fori_loop(0, _PPW, _flat1, 0)

    def _reduce_half(s, jlo):
        def _j(j, c):
            p_loc = s * _PPS + (j >> 4)
            r_ids = adjr1_v[j, :]
            pvec = jnp.full((16,), 0, jnp.int32) + p_loc
            sc = plsc.load_gather(ur_v, [pvec, r_ids])
            mx = jnp.max(sc)
            ex = jnp.exp(sc - mx)
            w = ex / jnp.sum(ex)
            w_v[...] = w
            acc_lo = jnp.zeros((16,), jnp.float32)
            acc_hi = jnp.zeros((16,), jnp.float32)
            for k in range(_NN):
                kvec = jnp.full((16,), k, jnp.int32)
                wk = plsc.load_gather(w_v, [kvec])
                row = j * _NN + k
                acc_lo = acc_lo + wk * e2_v[row, pl.ds(0, 16)]
                acc_hi = acc_hi + wk * e2_v[row, pl.ds(16, 16)]
            m1_v[j, pl.ds(0, 16)] = acc_lo
            m1_v[j, pl.ds(16, 16)] = acc_hi
            return c
        lax.fori_loop(jlo, jlo + _HH, _j, 0)

    def _sub(s, c):
        h1base = pbase * _NN + s * _H1S
        idxa = idx1_v.at[pl.ds(s * _H1S, _HH)]
        idxb = idx1_v.at[pl.ds(s * _H1S + _HH, _HH)]
        a1 = pltpu.async_copy(ent_h.at[idxa], e1_v.at[pl.ds(0, _HH)], sem)
        a2 = pltpu.async_copy(adj_e_h.at[idxa], adje1_v.at[pl.ds(0, _HH)], sem)
        a3 = pltpu.async_copy(adj_r_h.at[idxa], adjr1_v.at[pl.ds(0, _HH)], sem)
        b1 = pltpu.async_copy(ent_h.at[idxb], e1_v.at[pl.ds(_HH, _HH)], sem)
        b2 = pltpu.async_copy(adj_e_h.at[idxb], adje1_v.at[pl.ds(_HH, _HH)], sem)
        b3 = pltpu.async_copy(adj_r_h.at[idxb], adjr1_v.at[pl.ds(_HH, _HH)], sem)
        a2.wait()

        def _flat2a(t, cc):
            idx2_v[pl.ds(t * _NN, _NN)] = adje1_v[t, :]
            return cc
        lax.fori_loop(0, _HH, _flat2a, 0)
        e2a = pltpu.async_copy(ent_h.at[idx2_v.at[pl.ds(0, _HH * _NN)]],
                               e2_v.at[pl.ds(0, _HH * _NN)], sem)
        b2.wait()

        def _flat2b(t, cc):
            idx2_v[pl.ds(t * _NN, _NN)] = adje1_v[t, :]
            return cc
        lax.fori_loop(_HH, _H1S, _flat2b, 0)
        e2b = pltpu.async_copy(ent_h.at[idx2_v.at[pl.ds(_HH * _NN, _HH * _NN)]],
                               e2_v.at[pl.ds(_HH * _NN, _HH * _NN)], sem)
        a1.wait()
        a3.wait()
        b1.wait()
        b3.wait()
        pltpu.sync_copy(e1_v, e1_o.at[pl.ds(h1base, _H1S)])
        e2a.wait()
        _reduce_half(s, 0)
        e2b.wait()
        _reduce_half(s, _HH)
        pltpu.sync_copy(m1_v, m1_o.at[pl.ds(h1base, _H1S)])
        return c
    lax.fori_loop(0, _SUB, _sub, 0)


_sc_gather = pl.kernel(
    _sc_body,
    out_type=[
        jax.ShapeDtypeStruct((_B, _D), jnp.float32),
        jax.ShapeDtypeStruct((_B, _D), jnp.float32),
        jax.ShapeDtypeStruct((_B, _NN), jnp.int32),
        jax.ShapeDtypeStruct((_B * _NN, _D), jnp.float32),
        jax.ShapeDtypeStruct((_B * _NN, _D), jnp.float32),
    ],
    mesh=plsc.VectorSubcoreMesh(core_axis_name="c", subcore_axis_name="s"),
    compiler_params=pltpu.CompilerParams(use_tc_tiling_on_sc=False,
                                         needs_layout_passes=False),
    scratch_types=[
        pltpu.VMEM((_PPW,), jnp.int32),
        pltpu.VMEM((_PPW,), jnp.int32),
        pltpu.VMEM((_PPW, _D), jnp.float32),
        pltpu.VMEM((_PPW, _D), jnp.float32),
        pltpu.VMEM((_PPW, _NN), jnp.int32),
        pltpu.VMEM((_PPW, _NN), jnp.int32),
        pltpu.VMEM((_PPW, _NR), jnp.float32),
        pltpu.VMEM((_H1W,), jnp.int32),
        pltpu.VMEM((_H1S, _D), jnp.float32),
        pltpu.VMEM((_H1S, _NN), jnp.int32),
        pltpu.VMEM((_H1S, _NN), jnp.int32),
        pltpu.VMEM((_H2S,), jnp.int32),
        pltpu.VMEM((_H2S, _D), jnp.float32),
        pltpu.VMEM((_H1S, _D), jnp.float32),
        pltpu.VMEM((16,), jnp.float32),
        pltpu.SemaphoreType.DMA,
    ],
)

_URB = 4000


def _ur_body(u_ref, rel_ref, out_ref):
    out_ref[...] = lax.dot_general(u_ref[...], rel_ref[...],
                                   (((1,), (1,)), ((), ())),
                                   preferred_element_type=jnp.float32)


def _ur_call(user_emb, rel):
    return pl.pallas_call(
        _ur_body,
        grid=(_NU // _URB,),
        in_specs=[
            pl.BlockSpec((_URB, _D), lambda i: (i, 0)),
            pl.BlockSpec((_NR, _D), lambda i: (0, 0)),
        ],
        out_specs=pl.BlockSpec((_URB, _NR), lambda i: (i, 0)),
        out_shape=jax.ShapeDtypeStruct((_NU, _NR), jnp.float32),
    )(user_emb, rel)


_P = 512
_G = _B // _P


def _tc_body(u_ref, e0_ref, r0_ref, e1_ref, m1_ref,
             rel_ref, ww_ref, wb_ref, out_ref):
    u = u_ref[...]
    rel = rel_ref[...]
    ww = ww_ref[...]
    wb = wb_ref[...]
    ur = lax.dot_general(u, rel, (((1,), (1,)), ((), ())),
                         preferred_element_type=jnp.float32)

    r0 = r0_ref[...]
    s0 = jnp.zeros((_P, _NN), jnp.float32)
    for r in range(_NR):
        s0 = s0 + jnp.where(r0 == r, ur[:, r:r + 1], 0.0)
    w0 = jnp.exp(s0 - jnp.max(s0, axis=-1, keepdims=True))
    w0 = w0 / jnp.sum(w0, axis=-1, keepdims=True)

    e1 = e1_ref[...]
    m1 = m1_ref[...]
    h1 = jnp.maximum(
        lax.dot_general(e1 + m1, ww, (((1,), (1,)), ((), ())),
                        preferred_element_type=jnp.float32) + wb, 0.0)

    m0 = jnp.sum(w0[:, :, None] * e1.reshape(_P, _NN, _D), axis=1)
    e0 = e0_ref[...]
    h0 = jnp.maximum(
        lax.dot_general(e0 + m0, ww, (((1,), (1,)), ((), ())),
                        preferred_element_type=jnp.float32) + wb, 0.0)

    mo = jnp.sum(w0[:, :, None] * h1.reshape(_P, _NN, _D), axis=1)
    o = jnp.tanh(
        lax.dot_general(h0 + mo, ww, (((1,), (1,)), ((), ())),
                        preferred_element_type=jnp.float32) + wb)
    pred = jnp.sum(u * o, axis=1)
    out_ref[...] = jax.nn.sigmoid(pred).reshape(1, 1, _P)


def _tc_call(urows, e0, adjr0, e1, m1, rel, ww, wb2):
    return pl.pallas_call(
        _tc_body,
        grid=(_G,),
        in_specs=[
            pl.BlockSpec((_P, _D), lambda i: (i, 0)),
            pl.BlockSpec((_P, _D), lambda i: (i, 0)),
            pl.BlockSpec((_P, _NN), lambda i: (i, 0)),
            pl.BlockSpec((_P * _NN, _D), lambda i: (i, 0)),
            pl.BlockSpec((_P * _NN, _D), lambda i: (i, 0)),
            pl.BlockSpec((_NR, _D), lambda i: (0, 0)),
            pl.BlockSpec((_D, _D), lambda i: (0, 0)),
            pl.BlockSpec((1, _D), lambda i: (0, 0)),
        ],
        out_specs=pl.BlockSpec((1, 1, _P), lambda i: (i, 0, 0)),
        out_shape=jax.ShapeDtypeStruct((_G, 1, _P), jnp.float32),
    )(urows, e0, adjr0, e1, m1, rel, ww, wb2)


def kernel(pairs, adj_entity_np, adj_relation_np, entity_emb, relation_emb,
           user_emb, W_w, W_b):
    users = pairs[:, 0].astype(jnp.int32)
    items = pairs[:, 1].astype(jnp.int32)
    ur_all = _ur_call(user_emb, relation_emb)
    urows, e0, adjr0, e1, m1 = _sc_gather(
        users, items, adj_entity_np.astype(jnp.int32),
        adj_relation_np.astype(jnp.int32), entity_emb, user_emb, ur_all)
    out = _tc_call(urows, e0, adjr0, e1, m1,
                   relation_emb, W_w, W_b.reshape(1, _D))
    return out.reshape(_B)
